# Initial kernel scaffold; baseline (speedup 1.0000x reference)
import jax
import jax.numpy as jnp
from jax.experimental import pallas as pl


def kernel(xyz, density, xyz_min, xyz_max):
    raise NotImplementedError("write your pallas kernel here")


# speedup vs baseline: 8.8276x; 8.8276x over previous
import functools
import math

import jax
import jax.numpy as jnp
import numpy as _np
from jax import lax
from jax.experimental import pallas as pl
from jax.experimental.pallas import tpu as pltpu
from jax.experimental.pallas import tpu_sc as plsc

D = H = W = 256
N = 2097152
DHW = D * H * W

NC = 2
NS = 16
L = 16
NW = NC * NS
NPW = N // NW
B = 1024
GROUPS = B // L
CHUNKS = NPW // B
GCHUNK = 128
SUBB = B // GCHUNK
GPS = GCHUNK // L
GROWS = 8 * SUBB

_T = float(_np.float32(0.001))
_C = -2.0 * math.log1p(-_T)
_D_THRES = 4.0 + math.log(math.expm1(_C))

_mesh = plsc.VectorSubcoreMesh(
    core_axis_name="c", subcore_axis_name="s", num_cores=NC, num_subcores=NS)


@functools.partial(
    pl.kernel,
    out_type=jax.ShapeDtypeStruct((N,), jnp.int32),
    mesh=_mesh,
    scratch_types=[
        pltpu.VMEM((2, B), jnp.float32),
        pltpu.VMEM((2, B), jnp.float32),
        pltpu.VMEM((2, B), jnp.float32),
        pltpu.VMEM((B,), jnp.float32),
        pltpu.VMEM((B,), jnp.float32),
        pltpu.VMEM((B,), jnp.float32),
        pltpu.VMEM((GROWS, GCHUNK), jnp.int32),
        pltpu.VMEM((GROWS, GCHUNK), jnp.float32),
        pltpu.VMEM((B,), jnp.int32),
        pltpu.VMEM((6, L), jnp.float32),
        pltpu.SemaphoreType.DMA,
        pltpu.SemaphoreType.DMA,
    ],
)
def _sc_kernel(xs_hbm, ys_hbm, zs_hbm, dens_hbm, dens2d_hbm, sc_hbm, out_hbm,
               cx2, cy2, cz2, wx, wy, wz, idx, vals, mask, scales,
               sem_g, sem_c):
    wid = lax.axis_index("s") * NC + lax.axis_index("c")
    base = wid * NPW

    pltpu.sync_copy(sc_hbm, scales)
    sW = scales[0, :]
    sH = scales[1, :]
    sD = scales[2, :]
    oW = scales[3, :]
    oH = scales[4, :]
    oD = scales[5, :]
    thres = jnp.full((L,), _D_THRES, dtype=jnp.float32)
    one = jnp.full((L,), 1, jnp.int32)
    zero = jnp.full((L,), 0, jnp.int32)

    pltpu.async_copy(xs_hbm.at[pl.ds(base, B)], cx2.at[0], sem_c)
    pltpu.async_copy(ys_hbm.at[pl.ds(base, B)], cy2.at[0], sem_c)
    pltpu.async_copy(zs_hbm.at[pl.ds(base, B)], cz2.at[0], sem_c)

    def chunk_half(c, pb):
        cxp, cyp, czp = cx2.at[pb], cy2.at[pb], cz2.at[pb]
        nb = 1 - pb
        p0 = base + c * B

        pltpu.make_async_copy(xs_hbm.at[pl.ds(0, B)], cxp, sem_c).wait()
        pltpu.make_async_copy(xs_hbm.at[pl.ds(0, B)], cyp, sem_c).wait()
        pltpu.make_async_copy(xs_hbm.at[pl.ds(0, B)], czp, sem_c).wait()

        cn = jnp.minimum(c + 1, CHUNKS - 1)
        pn = base + cn * B
        pltpu.async_copy(xs_hbm.at[pl.ds(pn, B)], cx2.at[nb], sem_c)
        pltpu.async_copy(ys_hbm.at[pl.ds(pn, B)], cy2.at[nb], sem_c)
        pltpu.async_copy(zs_hbm.at[pl.ds(pn, B)], cz2.at[nb], sem_c)

        def gen_sub(s, _):
            r0 = s * 8
            for j in range(GPS):
                i = s * GPS + j
                xs = cxp[pl.ds(i * L, L)]
                ys = cyp[pl.ds(i * L, L)]
                zs = czp[pl.ds(i * L, L)]
                fW = zs * sW + oW
                fH = ys * sH + oH
                fD = xs * sD + oD
                x0 = jnp.clip(fW.astype(jnp.int32), 0, W - 2)
                y0 = jnp.clip(fH.astype(jnp.int32), 0, H - 2)
                z0 = jnp.clip(fD.astype(jnp.int32), 0, D - 2)
                wx[pl.ds(i * L, L)] = fW - x0.astype(jnp.float32)
                wy[pl.ds(i * L, L)] = fH - y0.astype(jnp.float32)
                wz[pl.ds(i * L, L)] = fD - z0.astype(jnp.float32)
                i000 = (z0 << 16) | (y0 << 8) | x0
                cc = j * L
                idx[r0 + 0, pl.ds(cc, L)] = i000
                idx[r0 + 1, pl.ds(cc, L)] = i000 + 1
                idx[r0 + 2, pl.ds(cc, L)] = i000 + W
                idx[r0 + 3, pl.ds(cc, L)] = i000 + (W + 1)
                idx[r0 + 4, pl.ds(cc, L)] = i000 + H * W
                idx[r0 + 5, pl.ds(cc, L)] = i000 + (H * W + 1)
                idx[r0 + 6, pl.ds(cc, L)] = i000 + (H * W + W)
                idx[r0 + 7, pl.ds(cc, L)] = i000 + (H * W + W + 1)
            for k in range(8):
                pltpu.async_copy(
                    dens_hbm.at[idx.at[r0 + k]], vals.at[r0 + k], sem_g)
            return 0

        lax.fori_loop(0, SUBB, gen_sub, 0)

        pltpu.make_async_copy(dens2d_hbm.at[pl.ds(0, GROWS), :],
                              vals, sem_g).wait()

        def combine(i, _):
            s = i * L
            r0 = (i >> 3) * 8
            cc = (i & 7) * L
            g000 = vals[r0 + 0, pl.ds(cc, L)]
            g001 = vals[r0 + 1, pl.ds(cc, L)]
            g010 = vals[r0 + 2, pl.ds(cc, L)]
            g011 = vals[r0 + 3, pl.ds(cc, L)]
            g100 = vals[r0 + 4, pl.ds(cc, L)]
            g101 = vals[r0 + 5, pl.ds(cc, L)]
            g110 = vals[r0 + 6, pl.ds(cc, L)]
            g111 = vals[r0 + 7, pl.ds(cc, L)]
            ax = wx[pl.ds(s, L)]
            ay = wy[pl.ds(s, L)]
            az = wz[pl.ds(s, L)]
            c00 = g000 + (g001 - g000) * ax
            c01 = g010 + (g011 - g010) * ax
            c10 = g100 + (g101 - g100) * ax
            c11 = g110 + (g111 - g110) * ax
            c0 = c00 + (c01 - c00) * ay
            c1 = c10 + (c11 - c10) * ay
            d = c0 + (c1 - c0) * az
            mask[pl.ds(s, L)] = jnp.where(d >= thres, one, zero)
            return 0

        lax.fori_loop(0, GROUPS, combine, 0)
        pltpu.sync_copy(mask, out_hbm.at[pl.ds(p0, B)])

    def pair_body(cc, _):
        chunk_half(2 * cc, 0)
        chunk_half(2 * cc + 1, 1)
        return 0

    lax.fori_loop(0, CHUNKS // 2, pair_body, 0)

    pltpu.make_async_copy(xs_hbm.at[pl.ds(0, B)], cx2.at[0], sem_c).wait()
    pltpu.make_async_copy(xs_hbm.at[pl.ds(0, B)], cy2.at[0], sem_c).wait()
    pltpu.make_async_copy(xs_hbm.at[pl.ds(0, B)], cz2.at[0], sem_c).wait()


def kernel(xyz, density, xyz_min, xyz_max):
    xs = xyz[:, 0]
    ys = xyz[:, 1]
    zs = xyz[:, 2]
    dens_flat = density.reshape(-1)
    dens2d = density.reshape(DHW // GCHUNK, GCHUNK)
    inv = 255.0 / (xyz_max - xyz_min)
    s = inv[::-1]
    o = (-xyz_min * inv)[::-1]
    sc = jnp.broadcast_to(
        jnp.concatenate([s, o]).reshape(6, 1).astype(jnp.float32), (6, L))
    m = _sc_kernel(xs, ys, zs, dens_flat, dens2d, sc)
    return m.astype(jnp.bool_)
